# Initial kernel scaffold; baseline (speedup 1.0000x reference)
import jax
import jax.numpy as jnp
from jax.experimental import pallas as pl


def kernel(theta, x_target, te_w0, te_b0, te_w1, te_b1, te_w2, te_b2, se_w0, se_b0, se_w1, se_b1, se_w2, se_b2, lm_w0, lm_b0, lm_w1, lm_b1, lm_w2, lm_b2):
    raise NotImplementedError("write your pallas kernel here")


# speedup vs baseline: 3.0930x; 3.0930x over previous
import functools

import jax
import jax.numpy as jnp
from jax import lax
from jax.experimental import pallas as pl
from jax.experimental.pallas import tpu as pltpu


def _theta_kernel(theta_ref, tw0, tb0, tw1, tb1, tw2, tb2,
                  sw2, sb2, wl0, bl0, tb_out, ws_out, bs_out):
    t = theta_ref[...]
    t = jnp.maximum(jnp.dot(t, tw0[...], preferred_element_type=jnp.float32)
                    + tb0[...], 0.0)
    t = jnp.maximum(jnp.dot(t, tw1[...], preferred_element_type=jnp.float32)
                    + tb1[...], 0.0)
    wl0_t = wl0[0:32, :]
    wl0_s = wl0[32:64, :]
    w_t = jnp.dot(tw2[...], wl0_t, preferred_element_type=jnp.float32)
    b_t = jnp.dot(tb2[...], wl0_t, preferred_element_type=jnp.float32) + bl0[...]
    tb_out[...] = (jnp.dot(t, w_t, preferred_element_type=jnp.float32) + b_t)
    ws_out[...] = jnp.dot(sw2[...], wl0_s, preferred_element_type=jnp.float32)
    bs_out[...] = jnp.dot(sb2[...], wl0_s, preferred_element_type=jnp.float32)


def _sim_kernel(nb, n_tgt, x_ref, tb_ref, w0, b0, w1, b1, ws, bs,
                lw1, lb1, lw2t, lb2, o_ref):
    r = nb * n_tgt
    x = x_ref[...].reshape(r, x_ref.shape[-1]).astype(jnp.bfloat16)
    h = jnp.dot(x, w0[...].astype(jnp.bfloat16),
                preferred_element_type=jnp.float32) + b0[...]
    h = jnp.maximum(h, 0.0).astype(jnp.bfloat16)
    h = jnp.dot(h, w1[...].astype(jnp.bfloat16),
                preferred_element_type=jnp.float32) + b1[...]
    h = jnp.maximum(h, 0.0).astype(jnp.bfloat16)
    h = jnp.dot(h, ws[...].astype(jnp.bfloat16),
                preferred_element_type=jnp.float32) + bs[...]
    h = h.reshape(nb, n_tgt, h.shape[-1]) + tb_ref[...]
    h = jnp.maximum(h, 0.0).reshape(r, h.shape[-1]).astype(jnp.bfloat16)
    h = jnp.dot(h, lw1[...].astype(jnp.bfloat16),
                preferred_element_type=jnp.float32) + lb1[...]
    h = jnp.maximum(h, 0.0).astype(jnp.bfloat16)
    row = lax.dot_general(lw2t[...].astype(jnp.bfloat16), h,
                          (((1,), (1,)), ((), ())),
                          preferred_element_type=jnp.float32) + lb2[...]
    o_ref[...] = row


def _rep(arr):
    zeros = (0,) * arr.ndim
    return pl.BlockSpec(arr.shape, lambda i: zeros)


def kernel(theta, x_target, te_w0, te_b0, te_w1, te_b1, te_w2, te_b2,
           se_w0, se_b0, se_w1, se_b1, se_w2, se_b2,
           lm_w0, lm_b0, lm_w1, lm_b1, lm_w2, lm_b2):
    B, theta_dim = theta.shape
    _, N, sim_dim = x_target.shape
    h0 = lm_w0.shape[1]

    tb, ws, bs = pl.pallas_call(
        _theta_kernel,
        out_shape=[
            jax.ShapeDtypeStruct((B, h0), jnp.float32),
            jax.ShapeDtypeStruct((se_w2.shape[0], h0), jnp.float32),
            jax.ShapeDtypeStruct((1, h0), jnp.float32),
        ],
        compiler_params=pltpu.CompilerParams(
            vmem_limit_bytes=64 * 1024 * 1024,
        ),
    )(theta, te_w0, te_b0, te_w1, te_b1, te_w2, te_b2,
      se_w2, se_b2, lm_w0, lm_b0)

    nb = 4
    while B % nb:
        nb //= 2
    grid = (B // nb,)
    lm_w2t = lm_w2.reshape(1, lm_w2.shape[0])
    tb3 = tb.reshape(B, 1, h0)

    in_specs = [
        pl.BlockSpec((nb, N, sim_dim), lambda i: (i, 0, 0)),
        pl.BlockSpec((nb, 1, h0), lambda i: (i, 0, 0)),
        _rep(se_w0), _rep(se_b0), _rep(se_w1), _rep(se_b1),
        _rep(ws), _rep(bs), _rep(lm_w1), _rep(lm_b1),
        _rep(lm_w2t), _rep(lm_b2),
    ]
    out = pl.pallas_call(
        functools.partial(_sim_kernel, nb, N),
        out_shape=jax.ShapeDtypeStruct((1, B * N), jnp.float32),
        grid=grid,
        in_specs=in_specs,
        out_specs=pl.BlockSpec((1, nb * N), lambda i: (0, i)),
        compiler_params=pltpu.CompilerParams(
            dimension_semantics=("parallel",),
            vmem_limit_bytes=64 * 1024 * 1024,
        ),
    )(x_target, tb3, se_w0, se_b0, se_w1, se_b1, ws, bs,
      lm_w1, lm_b1, lm_w2t, lm_b2)

    return out.reshape(B, N, 1)


# speedup vs baseline: 3.0943x; 1.0004x over previous
import functools

import jax
import jax.numpy as jnp
from jax import lax
from jax.experimental import pallas as pl
from jax.experimental.pallas import tpu as pltpu


def _theta_kernel(theta_ref, tw0, tb0, tw1, tb1, tw2, tb2,
                  sw2, sb2, wl0, bl0, tb_out, ws_out):
    t = theta_ref[...]
    t = jnp.maximum(jnp.dot(t, tw0[...], preferred_element_type=jnp.float32)
                    + tb0[...], 0.0)
    t = jnp.maximum(jnp.dot(t, tw1[...], preferred_element_type=jnp.float32)
                    + tb1[...], 0.0)
    wl0_t = wl0[0:32, :]
    wl0_s = wl0[32:64, :]
    w_t = jnp.dot(tw2[...], wl0_t, preferred_element_type=jnp.float32)
    b_t = (jnp.dot(tb2[...], wl0_t, preferred_element_type=jnp.float32)
           + jnp.dot(sb2[...], wl0_s, preferred_element_type=jnp.float32)
           + bl0[...])
    tb_out[...] = (jnp.dot(t, w_t, preferred_element_type=jnp.float32) + b_t)
    ws_out[...] = jnp.dot(sw2[...], wl0_s, preferred_element_type=jnp.float32)


def _sim_kernel(nb, n_tgt, x_ref, tb_ref, w0, b0, w1, b1, ws,
                lw1, lb1, lw2t, lb2, o_ref):
    r = nb * n_tgt
    x = x_ref[...].reshape(r, x_ref.shape[-1]).astype(jnp.bfloat16)
    h = jnp.dot(x, w0[...].astype(jnp.bfloat16),
                preferred_element_type=jnp.float32).astype(jnp.bfloat16)
    h = jnp.maximum(h + b0[...].astype(jnp.bfloat16), 0.0)
    h = jnp.dot(h, w1[...].astype(jnp.bfloat16),
                preferred_element_type=jnp.float32).astype(jnp.bfloat16)
    h = jnp.maximum(h + b1[...].astype(jnp.bfloat16), 0.0)
    h = jnp.dot(h, ws[...].astype(jnp.bfloat16),
                preferred_element_type=jnp.float32).astype(jnp.bfloat16)
    h = h.reshape(nb, n_tgt, h.shape[-1]) + tb_ref[...].astype(jnp.bfloat16)
    h = jnp.maximum(h, 0.0).reshape(r, h.shape[-1])
    h = jnp.dot(h, lw1[...].astype(jnp.bfloat16),
                preferred_element_type=jnp.float32).astype(jnp.bfloat16)
    h = jnp.maximum(h + lb1[...].astype(jnp.bfloat16), 0.0)
    row = lax.dot_general(lw2t[...].astype(jnp.bfloat16), h,
                          (((1,), (1,)), ((), ())),
                          preferred_element_type=jnp.float32) + lb2[...]
    o_ref[...] = row


def _rep(arr):
    zeros = (0,) * arr.ndim
    return pl.BlockSpec(arr.shape, lambda i: zeros)


def kernel(theta, x_target, te_w0, te_b0, te_w1, te_b1, te_w2, te_b2,
           se_w0, se_b0, se_w1, se_b1, se_w2, se_b2,
           lm_w0, lm_b0, lm_w1, lm_b1, lm_w2, lm_b2):
    B, theta_dim = theta.shape
    _, N, sim_dim = x_target.shape
    h0 = lm_w0.shape[1]

    tb, ws = pl.pallas_call(
        _theta_kernel,
        out_shape=[
            jax.ShapeDtypeStruct((B, h0), jnp.float32),
            jax.ShapeDtypeStruct((se_w2.shape[0], h0), jnp.float32),
        ],
        compiler_params=pltpu.CompilerParams(
            vmem_limit_bytes=64 * 1024 * 1024,
        ),
    )(theta, te_w0, te_b0, te_w1, te_b1, te_w2, te_b2,
      se_w2, se_b2, lm_w0, lm_b0)

    nb = 4
    while B % nb:
        nb //= 2
    grid = (B // nb,)
    lm_w2t = lm_w2.reshape(1, lm_w2.shape[0])
    tb3 = tb.reshape(B, 1, h0)

    in_specs = [
        pl.BlockSpec((nb, N, sim_dim), lambda i: (i, 0, 0)),
        pl.BlockSpec((nb, 1, h0), lambda i: (i, 0, 0)),
        _rep(se_w0), _rep(se_b0), _rep(se_w1), _rep(se_b1),
        _rep(ws), _rep(lm_w1), _rep(lm_b1),
        _rep(lm_w2t), _rep(lm_b2),
    ]
    out = pl.pallas_call(
        functools.partial(_sim_kernel, nb, N),
        out_shape=jax.ShapeDtypeStruct((1, B * N), jnp.float32),
        grid=grid,
        in_specs=in_specs,
        out_specs=pl.BlockSpec((1, nb * N), lambda i: (0, i)),
        compiler_params=pltpu.CompilerParams(
            dimension_semantics=("parallel",),
            vmem_limit_bytes=64 * 1024 * 1024,
        ),
    )(x_target, tb3, se_w0, se_b0, se_w1, se_b1, ws,
      lm_w1, lm_b1, lm_w2t, lm_b2)

    return out.reshape(B, N, 1)
